# baseline (speedup 1.0000x reference)
import jax
import jax.numpy as jnp
from jax import lax
from jax.experimental import pallas as pl
from jax.experimental.pallas import tpu as pltpu
from jax.experimental.pallas import tpu_sc as plsc

N_ROWS = 32768
N_CH = 256
N_SEG = 16
N_CORES = 2
N_SUBCORES = 16
LANES = 16

CH_HALF = N_CH // N_CORES
ROWS_PER_TILE = N_ROWS // N_SUBCORES
CHUNK = 128
N_CHUNKS = ROWS_PER_TILE // CHUNK
RING = 4
N_VECS = CH_HALF // LANES
GROUPS = CHUNK // LANES


def _rescale_body(feat_hbm, seg2d_hbm, noise_hbm, out_hbm,
                  ring, idx2d, pooled, noise_v, inv_v, acc,
                  in0, in1, in2, in3, ot0, ot1, ot2, ot3):
    in_sems = (in0, in1, in2, in3)
    out_sems = (ot0, ot1, ot2, ot3)
    c = lax.axis_index("c")
    s = lax.axis_index("s")
    ch0 = c * CH_HALF
    row0 = s * ROWS_PER_TILE

    def feat_src(k):
        return feat_hbm.at[pl.ds(row0 + k * CHUNK, CHUNK), pl.ds(ch0, CH_HALF)]

    def out_dst(k):
        return out_hbm.at[pl.ds(row0 + k * CHUNK, CHUNK), pl.ds(ch0, CH_HALF)]

    def slot(j):
        return ring.at[pl.ds(j * CHUNK, CHUNK)]

    def start_in(k, j):
        pltpu.async_copy(feat_src(k), slot(j), in_sems[j])

    def wait_in(j):
        pltpu.make_async_copy(feat_src(0), slot(j), in_sems[j]).wait()

    def start_out(k, j):
        pltpu.async_copy(slot(j), out_dst(k), out_sems[j])

    def wait_out(j):
        pltpu.make_async_copy(slot(j), out_dst(0), out_sems[j]).wait()

    pltpu.sync_copy(
        seg2d_hbm.at[pl.ds(s * (ROWS_PER_TILE // CHUNK), ROWS_PER_TILE // CHUNK)],
        idx2d)

    @pl.when(s == 0)
    def _():
        def zbody(seg, carry):
            for v in range(N_VECS):
                pooled[seg, pl.ds(v * LANES, LANES)] = jnp.zeros(
                    (LANES,), jnp.float32)
            return carry
        lax.fori_loop(0, N_SEG, zbody, 0)
        pltpu.sync_copy(pooled, acc)
    plsc.subcore_barrier()

    _p1scope = jax.named_scope("p1_segsum"); _p1scope.__enter__()
    start_in(0, 0)
    start_in(1, 1)

    def p1body(q, carry):
        for j in range(RING):
            k = q * RING + j
            wait_in(j)
            nj = (j + 2) % RING
            if j < 2:
                start_in(k + 2, nj)
            else:
                @pl.when(q < (N_CHUNKS // RING) - 1)
                def _():
                    start_in(k + 2, nj)
            pltpu.sync_copy(slot(j), acc.at[idx2d.at[k]], add=True)
        return carry
    lax.fori_loop(0, N_CHUNKS // RING, p1body, 0)
    plsc.subcore_barrier()
    _p1scope.__exit__(None, None, None)
    _p2scope = jax.named_scope("p2_inv"); _p2scope.__enter__()

    pltpu.sync_copy(acc, pooled)
    pltpu.sync_copy(noise_hbm.at[pl.ds(0, N_SEG), pl.ds(ch0, CH_HALF)], noise_v)

    def p2body(seg, carry):
        for v in range(N_VECS):
            p = pooled[seg, pl.ds(v * LANES, LANES)]
            nz = noise_v[seg, pl.ds(v * LANES, LANES)]
            inv_v[seg, pl.ds(v * LANES, LANES)] = 1.0 / ((0.875 + 0.25 * nz) * p)
        return carry
    lax.fori_loop(0, N_SEG, p2body, 0)
    _p2scope.__exit__(None, None, None)
    _p3scope = jax.named_scope("p3_rescale"); _p3scope.__enter__()

    def compute_chunk(k, j):
        base_j = j * CHUNK

        def gbody(g, gcarry):
            gi = k * CHUNK + g * LANES
            segvec = idx2d[lax.shift_right_logical(gi, 7),
                           pl.ds(lax.bitwise_and(gi, 127), LANES)]
            base = base_j + g * LANES
            for i in range(LANES):
                seg = segvec[i]
                for v in range(N_VECS):
                    col = v * LANES
                    ring[base + i, pl.ds(col, LANES)] = (
                        ring[base + i, pl.ds(col, LANES)]
                        * inv_v[seg, pl.ds(col, LANES)])
            return gcarry
        lax.fori_loop(0, GROUPS, gbody, 0)

    start_in(0, 0)
    start_in(1, 1)

    def p3body(q, carry):
        for j in range(RING):
            k = q * RING + j
            wait_in(j)
            compute_chunk(k, j)
            start_out(k, j)
            nj = (j + 2) % RING
            if j < 2:
                @pl.when(q > 0)
                def _():
                    wait_out(nj)
                start_in(k + 2, nj)
            else:
                @pl.when(q < (N_CHUNKS // RING) - 1)
                def _():
                    wait_out(nj)
                    start_in(k + 2, nj)
        return carry
    lax.fori_loop(0, N_CHUNKS // RING, p3body, 0)
    wait_out(0)
    wait_out(1)
    wait_out(2)
    wait_out(3)
    _p3scope.__exit__(None, None, None)


def kernel(features, segment_ids, rand_noise):
    seg2d = segment_ids.astype(jnp.int32).reshape(N_ROWS // CHUNK, CHUNK)
    mesh = plsc.VectorSubcoreMesh(core_axis_name="c", subcore_axis_name="s")
    run = pl.kernel(
        _rescale_body,
        mesh=mesh,
        out_type=jax.ShapeDtypeStruct((N_ROWS, N_CH), jnp.float32),
        scratch_types=[
            pltpu.VMEM((RING * CHUNK, CH_HALF), jnp.float32),
            pltpu.VMEM((ROWS_PER_TILE // CHUNK, CHUNK), jnp.int32),
            pltpu.VMEM((N_SEG, CH_HALF), jnp.float32),
            pltpu.VMEM((N_SEG, CH_HALF), jnp.float32),
            pltpu.VMEM((N_SEG, CH_HALF), jnp.float32),
            pltpu.VMEM_SHARED((N_SEG, CH_HALF), jnp.float32),
        ] + [pltpu.SemaphoreType.DMA] * 8,
    )
    return run(features, seg2d, rand_noise)


# speedup vs baseline: 1.9372x; 1.9372x over previous
import jax
import jax.numpy as jnp
from jax import lax
from jax.experimental import pallas as pl
from jax.experimental.pallas import tpu as pltpu
from jax.experimental.pallas import tpu_sc as plsc

N_ROWS = 32768
N_CH = 256
N_SEG = 16
N_CORES = 2
N_SUBCORES = 16
LANES = 16

CH_HALF = N_CH // N_CORES
ROWS_PER_TILE = N_ROWS // N_SUBCORES
CHUNK = 128
N_CHUNKS = ROWS_PER_TILE // CHUNK
RING = 4
N_VECS = CH_HALF // LANES


def _rescale_body(feat_hbm, seg2d_hbm, noise_hbm, out_hbm,
                  ring, sring, idx2d, pooled, noise_v, inv_v, acc,
                  in0, in1, in2, in3, ot0, ot1, ot2, ot3, gs0, gs1):
    in_sems = (in0, in1, in2, in3)
    out_sems = (ot0, ot1, ot2, ot3)
    g_sems = (gs0, gs1)
    c = lax.axis_index("c")
    s = lax.axis_index("s")
    ch0 = c * CH_HALF
    row0 = s * ROWS_PER_TILE

    def feat_src(k):
        return feat_hbm.at[pl.ds(row0 + k * CHUNK, CHUNK), pl.ds(ch0, CH_HALF)]

    def out_dst(k):
        return out_hbm.at[pl.ds(row0 + k * CHUNK, CHUNK), pl.ds(ch0, CH_HALF)]

    def slot(j):
        return ring.at[pl.ds(j * CHUNK, CHUNK)]

    def gslot(h):
        return sring.at[pl.ds(h * CHUNK, CHUNK)]

    def start_in(k, j):
        pltpu.async_copy(feat_src(k), slot(j), in_sems[j])

    def wait_in(j):
        pltpu.make_async_copy(feat_src(0), slot(j), in_sems[j]).wait()

    def start_out(k, j):
        pltpu.async_copy(slot(j), out_dst(k), out_sems[j])

    def wait_out(j):
        pltpu.make_async_copy(slot(j), out_dst(0), out_sems[j]).wait()

    def start_gather(k, h):
        pltpu.async_copy(acc.at[idx2d.at[k]], gslot(h), g_sems[h])

    def wait_gather(h):
        pltpu.make_async_copy(acc.at[idx2d.at[0]], gslot(h), g_sems[h]).wait()

    pltpu.sync_copy(
        seg2d_hbm.at[pl.ds(s * (ROWS_PER_TILE // CHUNK), ROWS_PER_TILE // CHUNK)],
        idx2d)

    @pl.when(s == 0)
    def _():
        def zbody(seg, carry):
            for v in range(N_VECS):
                pooled[seg, pl.ds(v * LANES, LANES)] = jnp.zeros(
                    (LANES,), jnp.float32)
            return carry
        lax.fori_loop(0, N_SEG, zbody, 0)
        pltpu.sync_copy(pooled, acc)
    plsc.subcore_barrier()

    start_in(0, 0)
    start_in(1, 1)

    def p1body(q, carry):
        for j in range(RING):
            k = q * RING + j
            wait_in(j)
            nj = (j + 2) % RING
            if j < 2:
                start_in(k + 2, nj)
            else:
                @pl.when(q < (N_CHUNKS // RING) - 1)
                def _():
                    start_in(k + 2, nj)
            pltpu.sync_copy(slot(j), acc.at[idx2d.at[k]], add=True)
        return carry
    lax.fori_loop(0, N_CHUNKS // RING, p1body, 0)
    plsc.subcore_barrier()

    @pl.when(s == 0)
    def _():
        pltpu.sync_copy(acc, pooled)
        pltpu.sync_copy(noise_hbm.at[pl.ds(0, N_SEG), pl.ds(ch0, CH_HALF)],
                        noise_v)

        def p2body(seg, carry):
            for v in range(N_VECS):
                p = pooled[seg, pl.ds(v * LANES, LANES)]
                nz = noise_v[seg, pl.ds(v * LANES, LANES)]
                inv_v[seg, pl.ds(v * LANES, LANES)] = (
                    1.0 / ((0.875 + 0.25 * nz) * p))
            return carry
        lax.fori_loop(0, N_SEG, p2body, 0)
        pltpu.sync_copy(inv_v, acc)
    plsc.subcore_barrier()

    start_in(0, 0)
    start_in(1, 1)
    start_gather(0, 0)
    start_gather(1, 1)

    def p3body(q, carry):
        for j in range(RING):
            k = q * RING + j
            h = j % 2
            wait_in(j)
            wait_gather(h)

            base_f = j * CHUNK
            base_g = h * CHUNK

            @plsc.parallel_loop(0, CHUNK, unroll=4)
            def mbody(r):
                for v in range(N_VECS):
                    col = v * LANES
                    ring[base_f + r, pl.ds(col, LANES)] = (
                        ring[base_f + r, pl.ds(col, LANES)]
                        * sring[base_g + r, pl.ds(col, LANES)])

            start_out(k, j)
            if j < 2:
                start_gather(k + 2, h)
                @pl.when(q > 0)
                def _():
                    wait_out((j + 2) % RING)
                start_in(k + 2, (j + 2) % RING)
            else:
                @pl.when(q < (N_CHUNKS // RING) - 1)
                def _():
                    start_gather(k + 2, h)
                    wait_out((j + 2) % RING)
                    start_in(k + 2, (j + 2) % RING)
        return carry
    lax.fori_loop(0, N_CHUNKS // RING, p3body, 0)
    wait_out(0)
    wait_out(1)
    wait_out(2)
    wait_out(3)


def kernel(features, segment_ids, rand_noise):
    seg2d = segment_ids.astype(jnp.int32).reshape(N_ROWS // CHUNK, CHUNK)
    mesh = plsc.VectorSubcoreMesh(core_axis_name="c", subcore_axis_name="s")
    run = pl.kernel(
        _rescale_body,
        mesh=mesh,
        out_type=jax.ShapeDtypeStruct((N_ROWS, N_CH), jnp.float32),
        scratch_types=[
            pltpu.VMEM((RING * CHUNK, CH_HALF), jnp.float32),
            pltpu.VMEM((2 * CHUNK, CH_HALF), jnp.float32),
            pltpu.VMEM((ROWS_PER_TILE // CHUNK, CHUNK), jnp.int32),
            pltpu.VMEM((N_SEG, CH_HALF), jnp.float32),
            pltpu.VMEM((N_SEG, CH_HALF), jnp.float32),
            pltpu.VMEM((N_SEG, CH_HALF), jnp.float32),
            pltpu.VMEM_SHARED((N_SEG, CH_HALF), jnp.float32),
        ] + [pltpu.SemaphoreType.DMA] * 10,
    )
    return run(features, seg2d, rand_noise)


# speedup vs baseline: 1.9471x; 1.0051x over previous
import jax
import jax.numpy as jnp
from jax import lax
from jax.experimental import pallas as pl
from jax.experimental.pallas import tpu as pltpu
from jax.experimental.pallas import tpu_sc as plsc

N_ROWS = 32768
N_CH = 256
N_SEG = 16
N_CORES = 2
N_SUBCORES = 16
LANES = 16

CH_HALF = N_CH // N_CORES
ROWS_PER_TILE = N_ROWS // N_SUBCORES
CHUNK = 128
N_CHUNKS = ROWS_PER_TILE // CHUNK
RING = 4
N_VECS = CH_HALF // LANES


def _rescale_body(feat_hbm, seg2d_hbm, noise_hbm, out_hbm,
                  ring, sring, idx2d, pooled, noise_v, inv_v, acc,
                  in0, in1, in2, in3, ot0, ot1, ot2, ot3, gs0, gs1):
    in_sems = (in0, in1, in2, in3)
    out_sems = (ot0, ot1, ot2, ot3)
    g_sems = (gs0, gs1)
    c = lax.axis_index("c")
    s = lax.axis_index("s")
    ch0 = c * CH_HALF
    row0 = s * ROWS_PER_TILE

    def feat_src(k):
        return feat_hbm.at[pl.ds(row0 + k * CHUNK, CHUNK), pl.ds(ch0, CH_HALF)]

    def out_dst(k):
        return out_hbm.at[pl.ds(row0 + k * CHUNK, CHUNK), pl.ds(ch0, CH_HALF)]

    def slot(j):
        return ring.at[pl.ds(j * CHUNK, CHUNK)]

    def gslot(h):
        return sring.at[pl.ds(h * CHUNK, CHUNK)]

    def start_in(k, j):
        pltpu.async_copy(feat_src(k), slot(j), in_sems[j])

    def wait_in(j):
        pltpu.make_async_copy(feat_src(0), slot(j), in_sems[j]).wait()

    def start_out(k, j):
        pltpu.async_copy(slot(j), out_dst(k), out_sems[j])

    def wait_out(j):
        pltpu.make_async_copy(slot(j), out_dst(0), out_sems[j]).wait()

    def start_gather(k, h):
        pltpu.async_copy(acc.at[idx2d.at[k]], gslot(h), g_sems[h])

    def wait_gather(h):
        pltpu.make_async_copy(acc.at[idx2d.at[0]], gslot(h), g_sems[h]).wait()

    pltpu.sync_copy(
        seg2d_hbm.at[pl.ds(s * (ROWS_PER_TILE // CHUNK), ROWS_PER_TILE // CHUNK)],
        idx2d)

    @pl.when(s == 0)
    def _():
        def zbody(seg, carry):
            for v in range(N_VECS):
                pooled[seg, pl.ds(v * LANES, LANES)] = jnp.zeros(
                    (LANES,), jnp.float32)
            return carry
        lax.fori_loop(0, N_SEG, zbody, 0)
        pltpu.sync_copy(pooled, acc)
    plsc.subcore_barrier()

    _p1 = jax.named_scope("p1_segsum"); _p1.__enter__()
    start_in(0, 0)
    start_in(1, 1)

    def p1body(q, carry):
        for j in range(RING):
            k = q * RING + j
            wait_in(j)
            nj = (j + 2) % RING
            if j < 2:
                start_in(k + 2, nj)
            else:
                @pl.when(q < (N_CHUNKS // RING) - 1)
                def _():
                    start_in(k + 2, nj)
            pltpu.sync_copy(slot(j), acc.at[idx2d.at[k]], add=True)
        return carry
    lax.fori_loop(0, N_CHUNKS // RING, p1body, 0)
    plsc.subcore_barrier()
    _p1.__exit__(None, None, None)
    _p2 = jax.named_scope("p2_inv"); _p2.__enter__()

    @pl.when(s == 0)
    def _():
        pltpu.sync_copy(acc, pooled)
        pltpu.sync_copy(noise_hbm.at[pl.ds(0, N_SEG), pl.ds(ch0, CH_HALF)],
                        noise_v)

        def p2body(seg, carry):
            for v in range(N_VECS):
                p = pooled[seg, pl.ds(v * LANES, LANES)]
                nz = noise_v[seg, pl.ds(v * LANES, LANES)]
                inv_v[seg, pl.ds(v * LANES, LANES)] = (
                    1.0 / ((0.875 + 0.25 * nz) * p))
            return carry
        lax.fori_loop(0, N_SEG, p2body, 0)
        pltpu.sync_copy(inv_v, acc)
    plsc.subcore_barrier()
    _p2.__exit__(None, None, None)
    _p3 = jax.named_scope("p3_rescale"); _p3.__enter__()

    start_in(0, 0)
    start_in(1, 1)
    start_gather(0, 0)
    start_gather(1, 1)

    def p3body(q, carry):
        for j in range(RING):
            k = q * RING + j
            h = j % 2
            wait_in(j)
            wait_gather(h)

            base_f = j * CHUNK
            base_g = h * CHUNK

            @plsc.parallel_loop(0, CHUNK, unroll=4)
            def mbody(r):
                for v in range(N_VECS):
                    col = v * LANES
                    ring[base_f + r, pl.ds(col, LANES)] = (
                        ring[base_f + r, pl.ds(col, LANES)]
                        * sring[base_g + r, pl.ds(col, LANES)])

            start_out(k, j)
            if j < 2:
                start_gather(k + 2, h)
                @pl.when(q > 0)
                def _():
                    wait_out((j + 2) % RING)
                start_in(k + 2, (j + 2) % RING)
            else:
                @pl.when(q < (N_CHUNKS // RING) - 1)
                def _():
                    start_gather(k + 2, h)
                    wait_out((j + 2) % RING)
                    start_in(k + 2, (j + 2) % RING)
        return carry
    lax.fori_loop(0, N_CHUNKS // RING, p3body, 0)
    wait_out(0)
    wait_out(1)
    wait_out(2)
    wait_out(3)
    _p3.__exit__(None, None, None)


def kernel(features, segment_ids, rand_noise):
    seg2d = segment_ids.astype(jnp.int32).reshape(N_ROWS // CHUNK, CHUNK)
    mesh = plsc.VectorSubcoreMesh(core_axis_name="c", subcore_axis_name="s")
    run = pl.kernel(
        _rescale_body,
        mesh=mesh,
        out_type=jax.ShapeDtypeStruct((N_ROWS, N_CH), jnp.float32),
        scratch_types=[
            pltpu.VMEM((RING * CHUNK, CH_HALF), jnp.float32),
            pltpu.VMEM((2 * CHUNK, CH_HALF), jnp.float32),
            pltpu.VMEM((ROWS_PER_TILE // CHUNK, CHUNK), jnp.int32),
            pltpu.VMEM((N_SEG, CH_HALF), jnp.float32),
            pltpu.VMEM((N_SEG, CH_HALF), jnp.float32),
            pltpu.VMEM((N_SEG, CH_HALF), jnp.float32),
            pltpu.VMEM_SHARED((N_SEG, CH_HALF), jnp.float32),
        ] + [pltpu.SemaphoreType.DMA] * 10,
    )
    return run(features, seg2d, rand_noise)
